# baseline (speedup 1.0000x reference)
import functools

import jax
import jax.numpy as jnp
from jax import lax
from jax.experimental import pallas as pl
from jax.experimental.pallas import tpu as pltpu
from jax.experimental.pallas import tpu_sc as plsc

B = 16384
D = 16

_NC, _NS = 2, 16
_NW = _NC * _NS
_BPW = B // _NW

@functools.cache
def _build_gather4():
    mesh = plsc.VectorSubcoreMesh(core_axis_name="c", subcore_axis_name="s")

    @functools.partial(
        pl.kernel,
        mesh=mesh,
        out_type=[jax.ShapeDtypeStruct((B, D), jnp.float32)] * 4,
        scratch_types=[
            pltpu.VMEM((_BPW,), jnp.int32),
            pltpu.VMEM((_BPW,), jnp.int32),
            pltpu.VMEM((_BPW, D), jnp.float32),
            pltpu.VMEM((_BPW, D), jnp.float32),
            pltpu.VMEM((_BPW, D), jnp.float32),
            pltpu.VMEM((_BPW, D), jnp.float32),
            pltpu.SemaphoreType.DMA,
        ],
        compiler_params=pltpu.CompilerParams(use_tc_tiling_on_sc=False),
    )
    def gather4(sid_hbm, pid_hbm, esg, epg, esm, epm,
                o_sg, o_pg, o_sm, o_pm,
                sidv, pidv, r_sg, r_pg, r_sm, r_pm, sem):
        wid = lax.axis_index("s") * _NC + lax.axis_index("c")
        base = wid * _BPW
        pltpu.sync_copy(sid_hbm.at[pl.ds(base, _BPW)], sidv)
        pltpu.sync_copy(pid_hbm.at[pl.ds(base, _BPW)], pidv)
        c1 = pltpu.async_copy(esg.at[sidv], r_sg, sem)
        c2 = pltpu.async_copy(epg.at[pidv], r_pg, sem)
        c3 = pltpu.async_copy(esm.at[sidv], r_sm, sem)
        c4 = pltpu.async_copy(epm.at[pidv], r_pm, sem)
        c1.wait()
        c2.wait()
        c3.wait()
        c4.wait()
        pltpu.sync_copy(r_sg, o_sg.at[pl.ds(base, _BPW)])
        pltpu.sync_copy(r_pg, o_pg.at[pl.ds(base, _BPW)])
        pltpu.sync_copy(r_sm, o_sm.at[pl.ds(base, _BPW)])
        pltpu.sync_copy(r_pm, o_pm.at[pl.ds(base, _BPW)])

    return gather4


def _mlp_body(sg, pg, sm, pm, w1a, w1b, b1, w2, b2, woh, wog, bo, out):
    gmf = sg[...] * pg[...]
    h1 = jnp.maximum(
        sm[...] @ w1a[...] + pm[...] @ w1b[...] + b1[...], 0.0)
    h2 = jnp.maximum(h1 @ w2[...] + b2[...], 0.0)
    z = (jnp.sum(h2 * woh[...], axis=1, keepdims=True)
         + jnp.sum(gmf * wog[...], axis=1, keepdims=True)
         + bo[...])
    out[...] = jnp.maximum(z, 0.0)


_BLK = 2048


def _mlp(sg, pg, sm, pm, w1a, w1b, b1, w2, b2, woh, wog, bo, interpret=False):
    row = lambda i: (i, 0)
    full = lambda i: (0, 0)
    return pl.pallas_call(
        _mlp_body,
        grid=(B // _BLK,),
        in_specs=[
            pl.BlockSpec((_BLK, D), row),
            pl.BlockSpec((_BLK, D), row),
            pl.BlockSpec((_BLK, D), row),
            pl.BlockSpec((_BLK, D), row),
            pl.BlockSpec((D, 32), full),
            pl.BlockSpec((D, 32), full),
            pl.BlockSpec((1, 32), full),
            pl.BlockSpec((32, D), full),
            pl.BlockSpec((1, D), full),
            pl.BlockSpec((1, D), full),
            pl.BlockSpec((1, D), full),
            pl.BlockSpec((1, 1), full),
        ],
        out_specs=pl.BlockSpec((_BLK, 1), row),
        out_shape=jax.ShapeDtypeStruct((B, 1), jnp.float32),
        interpret=interpret,
    )(sg, pg, sm, pm, w1a, w1b, b1, w2, b2, woh, wog, bo)


def kernel(sid, pid, E_sg, E_pg, E_sm, E_pm, W1, b1, W2, b2, Wo, bo):
    sg, pg, sm, pm = _build_gather4()(sid.astype(jnp.int32),
                                      pid.astype(jnp.int32),
                                      E_sg, E_pg, E_sm, E_pm)
    w1a = W1[:D]
    w1b = W1[D:]
    woh = Wo[:D].reshape(1, D)
    wog = Wo[D:].reshape(1, D)
    out = _mlp(sg, pg, sm, pm, w1a, w1b, b1.reshape(1, 32), W2,
               b2.reshape(1, D), woh, wog, bo.reshape(1, 1))
    return out.reshape(B)
